# Initial kernel scaffold; baseline (speedup 1.0000x reference)
import jax
import jax.numpy as jnp
from jax.experimental import pallas as pl


def kernel(query, g_prompt, e_prompt_pool, e_prompt_keys):
    raise NotImplementedError("write your pallas kernel here")


# speedup vs baseline: 4.2007x; 4.2007x over previous
import jax
import jax.numpy as jnp
from jax.experimental import pallas as pl
from jax.experimental.pallas import tpu as pltpu

B = 64
D = 768
H = 12
HD = 64
NG = 6
NE = 6
G_LEN = 5
E_LEN = 5
POOL = 64
TOPK = 8


def _topk_kernel(q_ref, k_ref, idx_ref):
    q = q_ref[...]
    k = k_ref[...]
    qn = q / jnp.maximum(jnp.sqrt(jnp.sum(q * q, axis=1, keepdims=True)), 1e-12)
    kn = k / jnp.maximum(jnp.sqrt(jnp.sum(k * k, axis=1, keepdims=True)), 1e-12)
    sim = jnp.dot(qn, kn.T)
    col = jax.lax.broadcasted_iota(jnp.int32, (B, POOL), 1)
    for t in range(TOPK):
        m = jnp.max(sim, axis=1, keepdims=True)
        amax = jnp.min(jnp.where(sim == m, col, POOL), axis=1)
        idx_ref[:, t] = amax
        sim = jnp.where(col == amax[:, None], -jnp.inf, sim)


def _gather_kernel(idx_ref, pool_ref, g_ref, ek_ref, ev_ref, gk_ref, gv_ref):
    b = pl.program_id(0)
    for t in range(TOPK):
        i = idx_ref[b, t]
        for e in range(E_LEN):
            ek_ref[:, 0, :, t * E_LEN + e, :] = pool_ref[i, :, 0, e, :, :]
            ev_ref[:, 0, :, t * E_LEN + e, :] = pool_ref[i, :, 1, e, :, :]
    for e in range(G_LEN):
        gk_ref[:, 0, :, e, :] = g_ref[:, 0, e, :, :]
        gv_ref[:, 0, :, e, :] = g_ref[:, 1, e, :, :]


def kernel(query, g_prompt, e_prompt_pool, e_prompt_keys):
    idx = pl.pallas_call(
        _topk_kernel,
        out_shape=jax.ShapeDtypeStruct((B, TOPK), jnp.int32),
    )(query, e_prompt_keys)

    pool_shape = e_prompt_pool.shape
    g_shape = g_prompt.shape

    ek, ev, gk, gv = pl.pallas_call(
        _gather_kernel,
        grid_spec=pltpu.PrefetchScalarGridSpec(
            num_scalar_prefetch=1,
            grid=(B,),
            in_specs=[
                pl.BlockSpec(pool_shape, lambda b, idx: (0, 0, 0, 0, 0, 0)),
                pl.BlockSpec(g_shape, lambda b, idx: (0, 0, 0, 0, 0)),
            ],
            out_specs=[
                pl.BlockSpec((NE, 1, H, TOPK * E_LEN, HD), lambda b, idx: (0, b, 0, 0, 0)),
                pl.BlockSpec((NE, 1, H, TOPK * E_LEN, HD), lambda b, idx: (0, b, 0, 0, 0)),
                pl.BlockSpec((NG, 1, H, G_LEN, HD), lambda b, idx: (0, b, 0, 0, 0)),
                pl.BlockSpec((NG, 1, H, G_LEN, HD), lambda b, idx: (0, b, 0, 0, 0)),
            ],
        ),
        out_shape=[
            jax.ShapeDtypeStruct((NE, B, H, TOPK * E_LEN, HD), jnp.float32),
            jax.ShapeDtypeStruct((NE, B, H, TOPK * E_LEN, HD), jnp.float32),
            jax.ShapeDtypeStruct((NG, B, H, G_LEN, HD), jnp.float32),
            jax.ShapeDtypeStruct((NG, B, H, G_LEN, HD), jnp.float32),
        ],
    )(idx, e_prompt_pool, g_prompt)
    return gk, gv, ek, ev
